# Initial kernel scaffold; baseline (speedup 1.0000x reference)
import jax
import jax.numpy as jnp
from jax.experimental import pallas as pl


def kernel(syn0, syn1, inputs, labels, sampled, batch_size, unigram_counts, negatives):
    raise NotImplementedError("write your pallas kernel here")


# speedup vs baseline: 1.4155x; 1.4155x over previous
import functools

import jax
import jax.numpy as jnp
from jax import lax
from jax.experimental import pallas as pl
from jax.experimental.pallas import tpu as pltpu
from jax.experimental.pallas import tpu_sc as plsc


def _softplus(v):
    e = jnp.exp(-jnp.abs(v))
    z = e / (e + 2.0)
    z2 = z * z
    p = z * (2.0 + z2 * (2.0 / 3.0 + z2 * (2.0 / 5.0 + z2 * (2.0 / 7.0
             + z2 * (2.0 / 9.0 + z2 * (2.0 / 11.0))))))
    return jnp.maximum(v, 0.0) + p


@functools.lru_cache(maxsize=None)
def _build(V, H, B, NEG):
    info = plsc.get_sparse_core_info()
    NC, NS, L = info.num_cores, info.num_subcores, info.num_lanes
    NW = NC * NS
    assert B % NW == 0
    TPW = B // NW
    T = 64
    assert TPW % T == 0 and T % L == 0 and H % 8 == 0
    NCHUNK = TPW // T
    NOUT = 1 + NEG

    mesh = plsc.VectorSubcoreMesh(core_axis_name="c", subcore_axis_name="s")

    @functools.partial(
        pl.kernel,
        mesh=mesh,
        compiler_params=pltpu.CompilerParams(needs_layout_passes=False),
        out_type=jax.ShapeDtypeStruct((B, NOUT), jnp.float32),
        scratch_types=[
            pltpu.VMEM((T,), jnp.int32),
            pltpu.VMEM((T,), jnp.int32),
            pltpu.VMEM((NEG, T), jnp.int32),
            pltpu.VMEM((T, H), jnp.float32),
            pltpu.VMEM((T, H), jnp.float32),
            pltpu.VMEM((NEG, T, H), jnp.float32),
            pltpu.VMEM((T, NOUT), jnp.float32),
            pltpu.SemaphoreType.DMA,
        ],
    )
    def k(syn0_h, syn1_h, inputs_h, labels_h, sampled_h, out_h,
          idx_in, idx_lab, idx_neg, in_rows, lab_rows, neg_rows,
          out_stage, sem):
        wid = lax.axis_index("s") * NC + lax.axis_index("c")
        wbase = wid * TPW
        iota = lax.iota(jnp.int32, L)
        nsplat = [jnp.full((L,), n, jnp.int32) for n in range(NEG)]
        csplat = [jnp.full((L,), j, jnp.int32) for j in range(NOUT)]
        zero = jnp.zeros((L,), jnp.float32)

        for c in range(NCHUNK):
            base = wbase + c * T
            pltpu.sync_copy(inputs_h.at[pl.ds(base, T)], idx_in)
            pltpu.sync_copy(labels_h.at[pl.ds(base, T)], idx_lab)
            for n in range(NEG):
                pltpu.sync_copy(sampled_h.at[n, pl.ds(base, T)], idx_neg.at[n])
            hs = [pltpu.async_copy(syn0_h.at[idx_in], in_rows, sem),
                  pltpu.async_copy(syn1_h.at[idx_lab], lab_rows, sem)]
            for n in range(NEG):
                hs.append(pltpu.async_copy(
                    syn1_h.at[idx_neg.at[n]], neg_rows.at[n], sem))
            for h in hs:
                h.wait()

            def tg_body(tg, carry):
                tvec = tg * L + iota

                def h_body(i, accs):
                    for dh in range(8):
                        hh = i * 8 + dh
                        hsplat = jnp.full((L,), 0, jnp.int32) + hh
                        inv = plsc.load_gather(in_rows, [tvec, hsplat])
                        lv = plsc.load_gather(lab_rows, [tvec, hsplat])
                        new = [accs[0] + inv * lv]
                        for n in range(NEG):
                            nv = plsc.load_gather(
                                neg_rows, [nsplat[n], tvec, hsplat])
                            new.append(accs[n + 1] + inv * nv)
                        accs = tuple(new)
                    return accs

                accs = lax.fori_loop(0, H // 8, h_body,
                                     tuple(zero for _ in range(NOUT)))
                plsc.store_scatter(out_stage, [tvec, csplat[0]],
                                   _softplus(-accs[0]))
                for n in range(NEG):
                    plsc.store_scatter(out_stage, [tvec, csplat[n + 1]],
                                       _softplus(accs[n + 1]))
                return carry

            lax.fori_loop(0, T // L, tg_body, 0)
            pltpu.sync_copy(out_stage, out_h.at[pl.ds(base, T)])

    return k


def kernel(syn0, syn1, inputs, labels, sampled, batch_size, unigram_counts,
           negatives):
    V, H = syn0.shape
    B, = inputs.shape
    NEG = sampled.shape[0]
    return _build(V, H, B, NEG)(syn0, syn1, inputs, labels, sampled)


# speedup vs baseline: 5.9516x; 4.2045x over previous
import functools

import jax
import jax.numpy as jnp
from jax import lax
from jax.experimental import pallas as pl
from jax.experimental.pallas import tpu as pltpu
from jax.experimental.pallas import tpu_sc as plsc


def _softplus(v):
    e = jnp.exp(-jnp.abs(v))
    z = e / (e + 2.0)
    z2 = z * z
    p = z * (2.0 + z2 * (2.0 / 3.0 + z2 * (2.0 / 5.0 + z2 * (2.0 / 7.0
             + z2 * (2.0 / 9.0 + z2 * (2.0 / 11.0))))))
    return jnp.maximum(v, 0.0) + p


@functools.lru_cache(maxsize=None)
def _build(V, H, B, NEG):
    info = plsc.get_sparse_core_info()
    NC, NS, L = info.num_cores, info.num_subcores, info.num_lanes
    NW = NC * NS
    assert B % NW == 0
    TPW = B // NW
    T = 64
    assert TPW % T == 0 and T % L == 0 and H % L == 0
    NCHUNK = TPW // T
    NOUT = 1 + NEG
    HV = H // L

    mesh = plsc.VectorSubcoreMesh(core_axis_name="c", subcore_axis_name="s")

    def _slot(shape, dtype):
        return [pltpu.VMEM(shape, dtype) for _ in range(2)]

    @functools.partial(
        pl.kernel,
        mesh=mesh,
        compiler_params=pltpu.CompilerParams(needs_layout_passes=False),
        out_type=jax.ShapeDtypeStruct((B * NOUT,), jnp.float32),
        scratch_types=[
            _slot((T,), jnp.int32),
            _slot((T,), jnp.int32),
            _slot((NEG, T), jnp.int32),
            _slot((T, H), jnp.float32),
            _slot((T, H), jnp.float32),
            _slot((NEG, T, H), jnp.float32),
            _slot((T * NOUT,), jnp.float32),
            [pltpu.SemaphoreType.DMA for _ in range(2)],
            [pltpu.SemaphoreType.DMA for _ in range(2)],
            [pltpu.SemaphoreType.DMA for _ in range(2)],
        ],
    )
    def k(syn0_h, syn1_h, inputs_h, labels_h, sampled_h, out_h,
          idx_in, idx_lab, idx_neg, in_rows, lab_rows, neg_rows,
          out_stage, gsem, isem, osem):
        wid = lax.axis_index("s") * NC + lax.axis_index("c")
        wbase = wid * TPW
        iota = lax.iota(jnp.int32, L)
        csplat = [jnp.full((L,), j, jnp.int32) for j in range(NOUT)]
        zero = jnp.zeros((L,), jnp.float32)

        def fire_idx(c):
            s = c & 1
            base = wbase + c * T
            hs = [
                pltpu.async_copy(inputs_h.at[pl.ds(base, T)], idx_in[s],
                                 isem[s]),
                pltpu.async_copy(labels_h.at[pl.ds(base, T)], idx_lab[s],
                                 isem[s]),
            ]
            for n in range(NEG):
                hs.append(pltpu.async_copy(
                    sampled_h.at[n, pl.ds(base, T)], idx_neg[s].at[n],
                    isem[s]))
            return hs

        def fire_rows(c):
            s = c & 1
            hs = [pltpu.async_copy(syn0_h.at[idx_in[s]], in_rows[s], gsem[s]),
                  pltpu.async_copy(syn1_h.at[idx_lab[s]], lab_rows[s],
                                   gsem[s])]
            for n in range(NEG):
                hs.append(pltpu.async_copy(
                    syn1_h.at[idx_neg[s].at[n]], neg_rows[s].at[n], gsem[s]))
            return hs

        def drain(hs):
            for h in hs:
                h.wait()

        def bcast_sum(v):
            for sh in (8, 4, 2, 1):
                v = v + jnp.take_along_axis(v, iota ^ sh, axis=0,
                                            mode="promise_in_bounds")
            return v

        def compute(c):
            s = c & 1
            inr, labr, negr, outs_ref = (in_rows[s], lab_rows[s], neg_rows[s],
                                         out_stage[s])

            def tg_body(tg, carry):
                t0 = tg * L

                def tok_body(tl, outs):
                    t = t0 + tl
                    inv = [inr[t, pl.ds(j * L, L)] for j in range(HV)]
                    accs = []
                    acc = zero
                    for j in range(HV):
                        acc = acc + inv[j] * labr[t, pl.ds(j * L, L)]
                    accs.append(acc)
                    for n in range(NEG):
                        acc = zero
                        for j in range(HV):
                            acc = acc + inv[j] * negr[n, t, pl.ds(j * L, L)]
                        accs.append(acc)
                    mask = iota == tl
                    return tuple(
                        jnp.where(mask, bcast_sum(a), o)
                        for a, o in zip(accs, outs))

                outs = lax.fori_loop(0, L, tok_body,
                                     tuple(zero for _ in range(NOUT)))
                tvec6 = (t0 + iota) * NOUT
                plsc.store_scatter(outs_ref, [tvec6 + csplat[0]],
                                   _softplus(-outs[0]))
                for n in range(NEG):
                    plsc.store_scatter(outs_ref, [tvec6 + csplat[n + 1]],
                                       _softplus(outs[n + 1]))
                return carry

            lax.fori_loop(0, T // L, tg_body, 0)

        idx_h = {0: fire_idx(0)}
        if NCHUNK > 1:
            idx_h[1] = fire_idx(1)
        drain(idx_h.pop(0))
        row_h = {0: fire_rows(0)}
        out_h_pend = {}
        for c in range(NCHUNK):
            if c + 1 < NCHUNK:
                drain(idx_h.pop(c + 1))
                row_h[c + 1] = fire_rows(c + 1)
            if c + 2 < NCHUNK:
                idx_h[c + 2] = fire_idx(c + 2)
            drain(row_h.pop(c))
            compute(c)
            s = c & 1
            if c >= 2:
                out_h_pend.pop(c - 2).wait()
            out_h_pend[c] = pltpu.async_copy(
                out_stage[s],
                out_h.at[pl.ds((wbase + c * T) * NOUT, T * NOUT)], osem[s])
        drain([h for _, h in sorted(out_h_pend.items())])

    return k


def kernel(syn0, syn1, inputs, labels, sampled, batch_size, unigram_counts,
           negatives):
    V, H = syn0.shape
    B, = inputs.shape
    NEG = sampled.shape[0]
    flat = _build(V, H, B, NEG)(syn0, syn1, inputs, labels, sampled)
    return flat.reshape(B, 1 + NEG)


# speedup vs baseline: 6.8094x; 1.1441x over previous
import functools

import jax
import jax.numpy as jnp
from jax import lax
from jax.experimental import pallas as pl
from jax.experimental.pallas import tpu as pltpu
from jax.experimental.pallas import tpu_sc as plsc


def _softplus(v):
    e = jnp.exp(-jnp.abs(v))
    z = e / (e + 2.0)
    z2 = z * z
    p = z * (2.0 + z2 * (2.0 / 3.0 + z2 * (2.0 / 5.0 + z2 * (2.0 / 7.0
             + z2 * (2.0 / 9.0 + z2 * (2.0 / 11.0))))))
    return jnp.maximum(v, 0.0) + p


@functools.lru_cache(maxsize=None)
def _build(V, H, B, NEG):
    info = plsc.get_sparse_core_info()
    NC, NS, L = info.num_cores, info.num_subcores, info.num_lanes
    NW = NC * NS
    assert B % NW == 0
    TPW = B // NW
    T = 64
    assert TPW % T == 0 and T % L == 0 and H % L == 0
    NCHUNK = TPW // T
    NOUT = 1 + NEG
    HV = H // L

    mesh = plsc.VectorSubcoreMesh(core_axis_name="c", subcore_axis_name="s")

    def _slot(shape, dtype):
        return [pltpu.VMEM(shape, dtype) for _ in range(2)]

    @functools.partial(
        pl.kernel,
        mesh=mesh,
        compiler_params=pltpu.CompilerParams(needs_layout_passes=False),
        out_type=jax.ShapeDtypeStruct((B, NOUT), jnp.float32),
        scratch_types=[
            _slot((T,), jnp.int32),
            _slot((T,), jnp.int32),
            _slot((NEG, T), jnp.int32),
            _slot((T, H), jnp.float32),
            _slot((T, H), jnp.float32),
            _slot((NEG, T, H), jnp.float32),
            pltpu.VMEM((T, NOUT), jnp.float32),
            [pltpu.SemaphoreType.DMA for _ in range(2)],
            [pltpu.SemaphoreType.DMA for _ in range(2)],
            pltpu.SemaphoreType.DMA,
        ],
    )
    def k(syn0_h, syn1_h, inputs_h, labels_h, sampled_h, out_h,
          idx_in, idx_lab, idx_neg, in_rows, lab_rows, neg_rows,
          out_stage, gsem, isem, osem):
        wid = lax.axis_index("s") * NC + lax.axis_index("c")
        wbase = wid * TPW
        iota = lax.iota(jnp.int32, L)
        csplat = [jnp.full((L,), j, jnp.int32) for j in range(NOUT)]
        zero = jnp.zeros((L,), jnp.float32)

        def fire_idx(c):
            s = c & 1
            base = wbase + c * T
            hs = [
                pltpu.async_copy(inputs_h.at[pl.ds(base, T)], idx_in[s],
                                 isem[s]),
                pltpu.async_copy(labels_h.at[pl.ds(base, T)], idx_lab[s],
                                 isem[s]),
            ]
            for n in range(NEG):
                hs.append(pltpu.async_copy(
                    sampled_h.at[n, pl.ds(base, T)], idx_neg[s].at[n],
                    isem[s]))
            return hs

        def fire_rows(c):
            s = c & 1
            hs = [pltpu.async_copy(syn0_h.at[idx_in[s]], in_rows[s], gsem[s]),
                  pltpu.async_copy(syn1_h.at[idx_lab[s]], lab_rows[s],
                                   gsem[s])]
            for n in range(NEG):
                hs.append(pltpu.async_copy(
                    syn1_h.at[idx_neg[s].at[n]], neg_rows[s].at[n], gsem[s]))
            return hs

        def drain(hs):
            for h in hs:
                h.wait()

        def bcast_sum(v):
            for sh in (8, 4, 2, 1):
                v = v + jnp.take_along_axis(v, iota ^ sh, axis=0,
                                            mode="promise_in_bounds")
            return v

        def compute(c):
            s = c & 1
            inr, labr, negr, outs_ref = (in_rows[s], lab_rows[s], neg_rows[s],
                                         out_stage)

            def tg_body(tg, carry):
                t0 = tg * L

                def tok_body(tl, outs):
                    t = t0 + tl
                    inv = [inr[t, pl.ds(j * L, L)] for j in range(HV)]
                    accs = []
                    acc = zero
                    for j in range(HV):
                        acc = acc + inv[j] * labr[t, pl.ds(j * L, L)]
                    accs.append(acc)
                    for n in range(NEG):
                        acc = zero
                        for j in range(HV):
                            acc = acc + inv[j] * negr[n, t, pl.ds(j * L, L)]
                        accs.append(acc)
                    mask = iota == tl
                    return tuple(
                        jnp.where(mask, bcast_sum(a), o)
                        for a, o in zip(accs, outs))

                outs = lax.fori_loop(0, L, tok_body,
                                     tuple(zero for _ in range(NOUT)))
                tvec = t0 + iota
                plsc.store_scatter(outs_ref, [tvec, csplat[0]],
                                   _softplus(-outs[0]))
                for n in range(NEG):
                    plsc.store_scatter(outs_ref, [tvec, csplat[n + 1]],
                                       _softplus(outs[n + 1]))
                return carry

            lax.fori_loop(0, T // L, tg_body, 0)

        idx_h = {0: fire_idx(0)}
        if NCHUNK > 1:
            idx_h[1] = fire_idx(1)
        drain(idx_h.pop(0))
        row_h = {0: fire_rows(0)}
        out_pend = None
        for c in range(NCHUNK):
            if c + 1 < NCHUNK:
                drain(idx_h.pop(c + 1))
                row_h[c + 1] = fire_rows(c + 1)
            if c + 2 < NCHUNK:
                idx_h[c + 2] = fire_idx(c + 2)
            drain(row_h.pop(c))
            if out_pend is not None:
                out_pend.wait()
            compute(c)
            out_pend = pltpu.async_copy(
                out_stage, out_h.at[pl.ds(wbase + c * T, T)], osem)
        out_pend.wait()

    return k


def kernel(syn0, syn1, inputs, labels, sampled, batch_size, unigram_counts,
           negatives):
    V, H = syn0.shape
    B, = inputs.shape
    NEG = sampled.shape[0]
    return _build(V, H, B, NEG)(syn0, syn1, inputs, labels, sampled)


# speedup vs baseline: 6.8163x; 1.0010x over previous
import functools

import jax
import jax.numpy as jnp
from jax import lax
from jax.experimental import pallas as pl
from jax.experimental.pallas import tpu as pltpu
from jax.experimental.pallas import tpu_sc as plsc


def _softplus(v):
    e = jnp.exp(-jnp.abs(v))
    z = e / (e + 2.0)
    z2 = z * z
    p = z * (2.0 + z2 * (2.0 / 3.0 + z2 * (2.0 / 5.0 + z2 * (2.0 / 7.0
             + z2 * (2.0 / 9.0 + z2 * (2.0 / 11.0))))))
    return jnp.maximum(v, 0.0) + p


@functools.lru_cache(maxsize=None)
def _build(V, H, B, NEG):
    info = plsc.get_sparse_core_info()
    NC, NS, L = info.num_cores, info.num_subcores, info.num_lanes
    NW = NC * NS
    assert B % NW == 0
    TPW = B // NW
    T = 64
    assert TPW % T == 0 and T % L == 0 and H % L == 0
    NCHUNK = TPW // T
    NOUT = 1 + NEG
    HV = H // L

    mesh = plsc.VectorSubcoreMesh(core_axis_name="c", subcore_axis_name="s")

    def _slot(shape, dtype):
        return [pltpu.VMEM(shape, dtype) for _ in range(2)]

    @functools.partial(
        pl.kernel,
        mesh=mesh,
        compiler_params=pltpu.CompilerParams(needs_layout_passes=False,
                                             use_tc_tiling_on_sc=True),
        out_type=jax.ShapeDtypeStruct((B, NOUT), jnp.float32),
        scratch_types=[
            _slot((T,), jnp.int32),
            _slot((T,), jnp.int32),
            _slot((NEG, T), jnp.int32),
            _slot((T, H), jnp.float32),
            _slot((T, H), jnp.float32),
            _slot((NEG, T, H), jnp.float32),
            pltpu.VMEM((T, NOUT), jnp.float32),
            [pltpu.SemaphoreType.DMA for _ in range(2)],
            [pltpu.SemaphoreType.DMA for _ in range(2)],
            pltpu.SemaphoreType.DMA,
        ],
    )
    def k(syn0_h, syn1_h, inputs_h, labels_h, sampled_h, out_h,
          idx_in, idx_lab, idx_neg, in_rows, lab_rows, neg_rows,
          out_stage, gsem, isem, osem):
        wid = lax.axis_index("s") * NC + lax.axis_index("c")
        wbase = wid * TPW
        iota = lax.iota(jnp.int32, L)
        csplat = [jnp.full((L,), j, jnp.int32) for j in range(NOUT)]
        zero = jnp.zeros((L,), jnp.float32)

        def fire_idx(c):
            s = c & 1
            base = wbase + c * T
            hs = [
                pltpu.async_copy(inputs_h.at[pl.ds(base, T)], idx_in[s],
                                 isem[s]),
                pltpu.async_copy(labels_h.at[pl.ds(base, T)], idx_lab[s],
                                 isem[s]),
            ]
            for n in range(NEG):
                hs.append(pltpu.async_copy(
                    sampled_h.at[n, pl.ds(base, T)], idx_neg[s].at[n],
                    isem[s]))
            return hs

        def fire_rows(c):
            s = c & 1
            hs = [pltpu.async_copy(syn0_h.at[idx_in[s]], in_rows[s], gsem[s]),
                  pltpu.async_copy(syn1_h.at[idx_lab[s]], lab_rows[s],
                                   gsem[s])]
            for n in range(NEG):
                hs.append(pltpu.async_copy(
                    syn1_h.at[idx_neg[s].at[n]], neg_rows[s].at[n], gsem[s]))
            return hs

        def drain(hs):
            for h in hs:
                h.wait()

        def bcast_sum(v):
            for sh in (8, 4, 2, 1):
                v = v + jnp.take_along_axis(v, iota ^ sh, axis=0,
                                            mode="promise_in_bounds")
            return v

        def compute(c):
            s = c & 1
            inr, labr, negr, outs_ref = (in_rows[s], lab_rows[s], neg_rows[s],
                                         out_stage)

            def tg_body(tg, carry):
                t0 = tg * L

                def tok_body(tl, outs):
                    t = t0 + tl
                    inv = [inr[t, pl.ds(j * L, L)] for j in range(HV)]
                    accs = []
                    acc = zero
                    for j in range(HV):
                        acc = acc + inv[j] * labr[t, pl.ds(j * L, L)]
                    accs.append(acc)
                    for n in range(NEG):
                        acc = zero
                        for j in range(HV):
                            acc = acc + inv[j] * negr[n, t, pl.ds(j * L, L)]
                        accs.append(acc)
                    mask = iota == tl
                    return tuple(
                        jnp.where(mask, bcast_sum(a), o)
                        for a, o in zip(accs, outs))

                outs = lax.fori_loop(0, L, tok_body,
                                     tuple(zero for _ in range(NOUT)))
                tvec = t0 + iota
                plsc.store_scatter(outs_ref, [tvec, csplat[0]],
                                   _softplus(-outs[0]))
                for n in range(NEG):
                    plsc.store_scatter(outs_ref, [tvec, csplat[n + 1]],
                                       _softplus(outs[n + 1]))
                return carry

            lax.fori_loop(0, T // L, tg_body, 0)

        idx_h = {0: fire_idx(0)}
        if NCHUNK > 1:
            idx_h[1] = fire_idx(1)
        drain(idx_h.pop(0))
        row_h = {0: fire_rows(0)}
        out_pend = None
        for c in range(NCHUNK):
            if c + 1 < NCHUNK:
                drain(idx_h.pop(c + 1))
                row_h[c + 1] = fire_rows(c + 1)
            if c + 2 < NCHUNK:
                idx_h[c + 2] = fire_idx(c + 2)
            drain(row_h.pop(c))
            if out_pend is not None:
                out_pend.wait()
            compute(c)
            out_pend = pltpu.async_copy(
                out_stage, out_h.at[pl.ds(wbase + c * T, T)], osem)
        out_pend.wait()

    return k


def kernel(syn0, syn1, inputs, labels, sampled, batch_size, unigram_counts,
           negatives):
    V, H = syn0.shape
    B, = inputs.shape
    NEG = sampled.shape[0]
    return _build(V, H, B, NEG)(syn0, syn1, inputs, labels, sampled)
